# baseline (speedup 1.0000x reference)
import functools

import jax
import jax.numpy as jnp
from jax import lax
from jax.experimental import pallas as pl
from jax.experimental.pallas import tpu as pltpu
from jax.experimental.pallas import tpu_sc as plsc

NUM_EXPERTS = 8
TOP_K = 2
HIDDEN = 1024
INTER = 1024
TOKENS = 2048

L = 16
NTILES = 16
TOK_PER_TILE = TOKENS // NTILES
GROUPS = TOK_PER_TILE // L
TM = 128
PAD_N = TOKENS * TOP_K + NUM_EXPERTS * TM
NT = PAD_N // TM
NT_PAD = 48
XCHUNKS = 4
CHUNK = TOK_PER_TILE * TOP_K // XCHUNKS

_f32 = jnp.float32
_i32 = jnp.int32


def _lanes():
    return lax.iota(_i32, L)


def _splat(vec, lane):
    return jnp.sum(jnp.where(_lanes() == lane, vec, 0))


def _route_body(logits_hbm, hidden_hbm,
                x_sorted_hbm, w_sorted_hbm, pos_hbm, te_hbm,
                lg_v, e_v, w_v, posk_v, slots_v, tok_v, wvals_v,
                cnt_v, allcnt_v, sm_v, te_v, rows_v,
                counts_sh, wsort_sh, sem):
    wid = lax.axis_index("s")
    base_t = wid * TOK_PER_TILE
    lanes = _lanes()

    pltpu.sync_copy(
        logits_hbm.at[pl.ds(base_t * NUM_EXPERTS, TOK_PER_TILE * NUM_EXPERTS)],
        lg_v)
    cnt = jnp.zeros((L,), _i32)
    for g in range(GROUPS):
        row_idx = (lanes + g * L) * NUM_EXPERTS
        cols = [plsc.load_gather(lg_v, [row_idx + e])
                for e in range(NUM_EXPERTS)]
        v1 = jnp.full((L,), -jnp.inf, _f32)
        e1 = jnp.zeros((L,), _i32)
        for e in range(NUM_EXPERTS):
            upd = cols[e] > v1
            v1 = jnp.where(upd, cols[e], v1)
            e1 = jnp.where(upd, e, e1)
        v2 = jnp.full((L,), -jnp.inf, _f32)
        e2 = jnp.zeros((L,), _i32)
        for e in range(NUM_EXPERTS):
            upd = (cols[e] > v2) & (e1 != e)
            v2 = jnp.where(upd, cols[e], v2)
            e2 = jnp.where(upd, e, e2)
        w1 = 1.0 / (1.0 + jnp.exp(v2 - v1))
        w2 = 1.0 - w1
        sl = pl.ds(g * L, L)
        e_v[0, sl] = e1
        e_v[1, sl] = e2
        w_v[0, sl] = w1
        w_v[1, sl] = w2
        for e in range(NUM_EXPERTS):
            pc = (plsc.all_reduce_population_count(e1 == e)
                  + plsc.all_reduce_population_count(e2 == e))
            cnt = cnt + jnp.where(lanes == e, pc, 0)

    cnt_v[...] = cnt
    pltpu.sync_copy(cnt_v, counts_sh.at[wid])
    plsc.subcore_barrier()

    pltpu.sync_copy(counts_sh, allcnt_v)
    total = jnp.zeros((L,), _i32)
    pre = jnp.zeros((L,), _i32)
    for t in range(NTILES):
        row = allcnt_v[t, :]
        total = total + row
        tv = jnp.full((L,), t, _i32)
        wv = jnp.zeros((L,), _i32) + wid
        pre = pre + jnp.where(tv < wv, row, 0)
    pcnt = (total + (TM - 1)) // TM * TM
    pcnt = jnp.where(lanes < NUM_EXPERTS, pcnt, 0)
    pbase = plsc.cumsum(pcnt) - pcnt
    sm_v[...] = pbase + pre

    run = jnp.zeros((L,), _i32)
    for g in range(GROUPS):
        sl = pl.ds(g * L, L)
        for k in range(TOP_K):
            a = e_v[k, sl]
            wt = w_v[k, sl]
            slot = jnp.zeros((L,), _i32)
            sr = sm_v[...] + run
            for e in range(NUM_EXPERTS):
                m = a == e
                r = plsc.cumsum(m.astype(_i32)) - 1
                base_e = _splat(sr, e)
                slot = jnp.where(m, base_e + r, slot)
                run = run + jnp.where(lanes == e, plsc.all_reduce_population_count(m), 0)
            posk_v[k, sl] = slot
            ai = g * (TOP_K * L) + k * L
            c, off = ai // CHUNK, ai % CHUNK
            slots_v[c, pl.ds(off, L)] = slot
            tok_v[c, pl.ds(off, L)] = base_t + g * L + lanes
            wvals_v[c, pl.ds(off, L)] = wt

    pltpu.sync_copy(posk_v.at[0], pos_hbm.at[pl.ds(base_t, TOK_PER_TILE)])
    pltpu.sync_copy(posk_v.at[1], pos_hbm.at[pl.ds(TOKENS + base_t, TOK_PER_TILE)])

    for c in range(XCHUNKS):
        pltpu.sync_copy(wvals_v.at[c], wsort_sh.at[slots_v.at[c]])
    plsc.subcore_barrier()

    @pl.when(wid == 0)
    def _write_meta():
        pltpu.sync_copy(wsort_sh, w_sorted_hbm)
        total_pad = jnp.sum(pcnt)
        for b in range(NT_PAD // L):
            j = lanes + b * L
            jtm = j * TM
            acc = jnp.zeros((L,), _i32)
            for e in range(1, NUM_EXPERTS):
                acc = acc + (jtm >= _splat(pbase, e)).astype(_i32)
            te = jnp.where(jtm < total_pad, acc, 0)
            te_v[pl.ds(b * L, L)] = te
        pltpu.sync_copy(te_v, te_hbm)

    for c in range(XCHUNKS):
        pltpu.async_copy(hidden_hbm.at[tok_v.at[c]], rows_v, sem).wait()
        pltpu.async_copy(rows_v, x_sorted_hbm.at[slots_v.at[c]], sem).wait()


def _route(router_logits, hidden_states):
    mesh = plsc.VectorSubcoreMesh(core_axis_name="c", subcore_axis_name="s",
                                  num_cores=1)
    kern = functools.partial(
        pl.kernel,
        out_type=(
            jax.ShapeDtypeStruct((PAD_N, HIDDEN), _f32),
            jax.ShapeDtypeStruct((PAD_N,), _f32),
            jax.ShapeDtypeStruct((TOP_K * TOKENS,), _i32),
            jax.ShapeDtypeStruct((NT_PAD,), _i32),
        ),
        mesh=mesh,
        scratch_types=[
            pltpu.VMEM((TOK_PER_TILE * NUM_EXPERTS,), _f32),
            pltpu.VMEM((TOP_K, TOK_PER_TILE), _i32),
            pltpu.VMEM((TOP_K, TOK_PER_TILE), _f32),
            pltpu.VMEM((TOP_K, TOK_PER_TILE), _i32),
            pltpu.VMEM((XCHUNKS, CHUNK), _i32),
            pltpu.VMEM((XCHUNKS, CHUNK), _i32),
            pltpu.VMEM((XCHUNKS, CHUNK), _f32),
            pltpu.VMEM((L,), _i32),
            pltpu.VMEM((NTILES, L), _i32),
            pltpu.VMEM((L,), _i32),
            pltpu.VMEM((NT_PAD,), _i32),
            pltpu.VMEM((CHUNK, HIDDEN), _f32),
            pltpu.VMEM_SHARED((NTILES, L), _i32),
            pltpu.VMEM_SHARED((PAD_N,), _f32),
            pltpu.SemaphoreType.DMA,
        ],
        compiler_params=pltpu.CompilerParams(needs_layout_passes=False),
    )(_route_body)
    return kern(router_logits, hidden_states)


def _gemm_body(te_ref, s1_ref, s2_ref, x_ref, w13_ref, w2_ref, wrow_ref,
               out_ref):
    j = pl.program_id(0)
    e = te_ref[j]
    x = x_ref[...]
    w13 = w13_ref[0]
    w2 = w2_ref[0]
    gateup = lax.dot_general(x, w13, (((1,), (1,)), ((), ())),
                             preferred_element_type=_f32)
    gateup = gateup * s1_ref[e]
    gate = gateup[:, :INTER]
    up = gateup[:, INTER:]
    act = gate * (1.0 / (1.0 + jnp.exp(-gate))) * up
    down = lax.dot_general(act, w2, (((1,), (1,)), ((), ())),
                           preferred_element_type=_f32)
    down = down * s2_ref[e]
    out_ref[...] = down * wrow_ref[0, 0, :][:, None]


def _gemm(te, s1, s2, x_sorted, w13_weight, w2_weight, w_rows):
    return pl.pallas_call(
        _gemm_body,
        grid_spec=pltpu.PrefetchScalarGridSpec(
            num_scalar_prefetch=3,
            grid=(NT,),
            in_specs=[
                pl.BlockSpec((TM, HIDDEN), lambda j, te, s1, s2: (j, 0)),
                pl.BlockSpec((1, 2 * INTER, HIDDEN),
                             lambda j, te, s1, s2: (te[j], 0, 0)),
                pl.BlockSpec((1, HIDDEN, INTER),
                             lambda j, te, s1, s2: (te[j], 0, 0)),
                pl.BlockSpec((1, 1, TM), lambda j, te, s1, s2: (j, 0, 0)),
            ],
            out_specs=pl.BlockSpec((TM, HIDDEN), lambda j, te, s1, s2: (j, 0)),
        ),
        out_shape=jax.ShapeDtypeStruct((PAD_N, HIDDEN), _f32),
        compiler_params=pltpu.CompilerParams(
            dimension_semantics=("arbitrary",),
        ),
    )(te, s1, s2, x_sorted, w13_weight, w2_weight, w_rows)


def _combine_body(down_hbm, pos_hbm, out_hbm, pa_v, pb_v, bufa_v, bufb_v, sem):
    wid = lax.axis_index("s")
    base_t = wid * TOK_PER_TILE
    ctok = CHUNK // TOP_K
    for c in range(XCHUNKS):
        pltpu.sync_copy(pos_hbm.at[pl.ds(base_t + c * ctok, ctok)], pa_v.at[c])
        pltpu.sync_copy(
            pos_hbm.at[pl.ds(TOKENS + base_t + c * ctok, ctok)], pb_v.at[c])
        pltpu.async_copy(down_hbm.at[pa_v.at[c]], bufa_v, sem).wait()
        pltpu.async_copy(down_hbm.at[pb_v.at[c]], bufb_v, sem).wait()

        def hb_body(hb, _):
            sl = pl.ds(hb * L, L)
            for i in range(ctok):
                bufa_v[i, sl] = bufa_v[i, sl] + bufb_v[i, sl]
            return 0

        lax.fori_loop(0, HIDDEN // L, hb_body, 0)
        pltpu.sync_copy(bufa_v, out_hbm.at[pl.ds(base_t + c * ctok, ctok)])


def _combine(down, pos):
    mesh = plsc.VectorSubcoreMesh(core_axis_name="c", subcore_axis_name="s",
                                  num_cores=1)
    ctok = CHUNK // TOP_K
    kern = functools.partial(
        pl.kernel,
        out_type=jax.ShapeDtypeStruct((TOKENS, HIDDEN), _f32),
        mesh=mesh,
        scratch_types=[
            pltpu.VMEM((XCHUNKS, ctok), _i32),
            pltpu.VMEM((XCHUNKS, ctok), _i32),
            pltpu.VMEM((ctok, HIDDEN), _f32),
            pltpu.VMEM((ctok, HIDDEN), _f32),
            pltpu.SemaphoreType.DMA,
        ],
        compiler_params=pltpu.CompilerParams(needs_layout_passes=False),
    )(_combine_body)
    return kern(down, pos)


def kernel(hidden_states, router_logits, w13_weight, w2_weight,
           w13_input_scale, w2_input_scale, w13_weight_scale, w2_weight_scale):
    s1 = (w13_input_scale * w13_weight_scale).astype(_f32)
    s2 = (w2_input_scale * w2_weight_scale).astype(_f32)
    x_sorted, w_sorted, pos, te = _route(
        router_logits.astype(_f32).reshape(TOKENS * NUM_EXPERTS),
        hidden_states.astype(_f32))
    w_rows = w_sorted.reshape(NT, 1, TM)
    down = _gemm(te, s1, s2, x_sorted, w13_weight, w2_weight, w_rows)
    return _combine(down, pos)


# speedup vs baseline: 1.2281x; 1.2281x over previous
import functools

import jax
import jax.numpy as jnp
from jax import lax
from jax.experimental import pallas as pl
from jax.experimental.pallas import tpu as pltpu
from jax.experimental.pallas import tpu_sc as plsc

NUM_EXPERTS = 8
TOP_K = 2
HIDDEN = 1024
INTER = 1024
TOKENS = 2048

L = 16
NTILES = 16
TOK_PER_TILE = TOKENS // NTILES
GROUPS = TOK_PER_TILE // L
TM = 256
PAD_N = TOKENS * TOP_K + NUM_EXPERTS * TM
NT = PAD_N // TM
NT_PAD = 32
XCHUNKS = 4
CHUNK = TOK_PER_TILE * TOP_K // XCHUNKS

_f32 = jnp.float32
_i32 = jnp.int32


def _lanes():
    return lax.iota(_i32, L)


def _splat(vec, lane):
    return jnp.sum(jnp.where(_lanes() == lane, vec, 0))


def _route_body(logits_hbm, hidden_hbm,
                x_sorted_hbm, w_sorted_hbm, pos_hbm, te_hbm,
                lg_v, e_v, w_v, posk_v, slots_v, tok_v, wvals_v,
                cnt_v, allcnt_v, sm_v, te_v, rows_v,
                counts_sh, wsort_sh, sem):
    wid = lax.axis_index("s")
    base_t = wid * TOK_PER_TILE
    lanes = _lanes()

    pltpu.sync_copy(
        logits_hbm.at[pl.ds(base_t * NUM_EXPERTS, TOK_PER_TILE * NUM_EXPERTS)],
        lg_v)
    cnt = jnp.zeros((L,), _i32)
    for g in range(GROUPS):
        row_idx = (lanes + g * L) * NUM_EXPERTS
        cols = [plsc.load_gather(lg_v, [row_idx + e])
                for e in range(NUM_EXPERTS)]
        v1 = jnp.full((L,), -jnp.inf, _f32)
        e1 = jnp.zeros((L,), _i32)
        for e in range(NUM_EXPERTS):
            upd = cols[e] > v1
            v1 = jnp.where(upd, cols[e], v1)
            e1 = jnp.where(upd, e, e1)
        v2 = jnp.full((L,), -jnp.inf, _f32)
        e2 = jnp.zeros((L,), _i32)
        for e in range(NUM_EXPERTS):
            upd = (cols[e] > v2) & (e1 != e)
            v2 = jnp.where(upd, cols[e], v2)
            e2 = jnp.where(upd, e, e2)
        w1 = 1.0 / (1.0 + jnp.exp(v2 - v1))
        w2 = 1.0 - w1
        sl = pl.ds(g * L, L)
        e_v[0, sl] = e1
        e_v[1, sl] = e2
        w_v[0, sl] = w1
        w_v[1, sl] = w2
        for e in range(NUM_EXPERTS):
            pc = (plsc.all_reduce_population_count(e1 == e)
                  + plsc.all_reduce_population_count(e2 == e))
            cnt = cnt + jnp.where(lanes == e, pc, 0)

    cnt_v[...] = cnt
    pltpu.sync_copy(cnt_v, counts_sh.at[wid])
    plsc.subcore_barrier()

    pltpu.sync_copy(counts_sh, allcnt_v)
    total = jnp.zeros((L,), _i32)
    pre = jnp.zeros((L,), _i32)
    for t in range(NTILES):
        row = allcnt_v[t, :]
        total = total + row
        tv = jnp.full((L,), t, _i32)
        wv = jnp.zeros((L,), _i32) + wid
        pre = pre + jnp.where(tv < wv, row, 0)
    pcnt = (total + (TM - 1)) // TM * TM
    pcnt = jnp.where(lanes < NUM_EXPERTS, pcnt, 0)
    pbase = plsc.cumsum(pcnt) - pcnt
    sm_v[...] = pbase + pre

    run = jnp.zeros((L,), _i32)
    for g in range(GROUPS):
        sl = pl.ds(g * L, L)
        for k in range(TOP_K):
            a = e_v[k, sl]
            wt = w_v[k, sl]
            slot = jnp.zeros((L,), _i32)
            sr = sm_v[...] + run
            for e in range(NUM_EXPERTS):
                m = a == e
                r = plsc.cumsum(m.astype(_i32)) - 1
                base_e = _splat(sr, e)
                slot = jnp.where(m, base_e + r, slot)
                run = run + jnp.where(lanes == e, plsc.all_reduce_population_count(m), 0)
            posk_v[k, sl] = slot
            ai = g * (TOP_K * L) + k * L
            c, off = ai // CHUNK, ai % CHUNK
            slots_v[c, pl.ds(off, L)] = slot
            tok_v[c, pl.ds(off, L)] = base_t + g * L + lanes
            wvals_v[c, pl.ds(off, L)] = wt

    pltpu.sync_copy(posk_v.at[0], pos_hbm.at[pl.ds(base_t, TOK_PER_TILE)])
    pltpu.sync_copy(posk_v.at[1], pos_hbm.at[pl.ds(TOKENS + base_t, TOK_PER_TILE)])

    for c in range(XCHUNKS):
        pltpu.sync_copy(wvals_v.at[c], wsort_sh.at[slots_v.at[c]])
    plsc.subcore_barrier()

    @pl.when(wid == 0)
    def _write_meta():
        pltpu.sync_copy(wsort_sh, w_sorted_hbm)
        total_pad = jnp.sum(pcnt)
        for b in range(NT_PAD // L):
            j = lanes + b * L
            jtm = j * TM
            acc = jnp.zeros((L,), _i32)
            for e in range(1, NUM_EXPERTS):
                acc = acc + (jtm >= _splat(pbase, e)).astype(_i32)
            te = jnp.where(jtm < total_pad, acc, 0)
            te_v[pl.ds(b * L, L)] = te
        pltpu.sync_copy(te_v, te_hbm)

    for c in range(XCHUNKS):
        pltpu.async_copy(hidden_hbm.at[tok_v.at[c]], rows_v, sem).wait()
        pltpu.async_copy(rows_v, x_sorted_hbm.at[slots_v.at[c]], sem).wait()


def _route(router_logits, hidden_states):
    mesh = plsc.VectorSubcoreMesh(core_axis_name="c", subcore_axis_name="s",
                                  num_cores=1)
    kern = functools.partial(
        pl.kernel,
        out_type=(
            jax.ShapeDtypeStruct((PAD_N, HIDDEN), _f32),
            jax.ShapeDtypeStruct((PAD_N,), _f32),
            jax.ShapeDtypeStruct((TOP_K * TOKENS,), _i32),
            jax.ShapeDtypeStruct((NT_PAD,), _i32),
        ),
        mesh=mesh,
        scratch_types=[
            pltpu.VMEM((TOK_PER_TILE * NUM_EXPERTS,), _f32),
            pltpu.VMEM((TOP_K, TOK_PER_TILE), _i32),
            pltpu.VMEM((TOP_K, TOK_PER_TILE), _f32),
            pltpu.VMEM((TOP_K, TOK_PER_TILE), _i32),
            pltpu.VMEM((XCHUNKS, CHUNK), _i32),
            pltpu.VMEM((XCHUNKS, CHUNK), _i32),
            pltpu.VMEM((XCHUNKS, CHUNK), _f32),
            pltpu.VMEM((L,), _i32),
            pltpu.VMEM((NTILES, L), _i32),
            pltpu.VMEM((L,), _i32),
            pltpu.VMEM((NT_PAD,), _i32),
            pltpu.VMEM((CHUNK, HIDDEN), _f32),
            pltpu.VMEM_SHARED((NTILES, L), _i32),
            pltpu.VMEM_SHARED((PAD_N,), _f32),
            pltpu.SemaphoreType.DMA,
        ],
        compiler_params=pltpu.CompilerParams(needs_layout_passes=False),
    )(_route_body)
    return kern(router_logits, hidden_states)


def _gemm_body(te_ref, s1_ref, s2_ref, x_ref, w13_ref, w2_ref, wrow_ref,
               out_ref):
    j = pl.program_id(0)
    e = te_ref[j]
    x = x_ref[...]
    w13 = w13_ref[0]
    w2 = w2_ref[0]
    gateup = lax.dot_general(x, w13, (((1,), (1,)), ((), ())),
                             preferred_element_type=_f32)
    gateup = gateup * s1_ref[e]
    gate = gateup[:, :INTER]
    up = gateup[:, INTER:]
    act = gate * (1.0 / (1.0 + jnp.exp(-gate))) * up
    down = lax.dot_general(act, w2, (((1,), (1,)), ((), ())),
                           preferred_element_type=_f32)
    down = down * s2_ref[e]
    out_ref[...] = down * wrow_ref[0, 0, :][:, None]


def _gemm(te, s1, s2, x_sorted, w13_weight, w2_weight, w_rows):
    return pl.pallas_call(
        _gemm_body,
        grid_spec=pltpu.PrefetchScalarGridSpec(
            num_scalar_prefetch=3,
            grid=(NT,),
            in_specs=[
                pl.BlockSpec((TM, HIDDEN), lambda j, te, s1, s2: (j, 0)),
                pl.BlockSpec((1, 2 * INTER, HIDDEN),
                             lambda j, te, s1, s2: (te[j], 0, 0)),
                pl.BlockSpec((1, HIDDEN, INTER),
                             lambda j, te, s1, s2: (te[j], 0, 0)),
                pl.BlockSpec((1, 1, TM), lambda j, te, s1, s2: (j, 0, 0)),
            ],
            out_specs=pl.BlockSpec((TM, HIDDEN), lambda j, te, s1, s2: (j, 0)),
        ),
        out_shape=jax.ShapeDtypeStruct((PAD_N, HIDDEN), _f32),
        compiler_params=pltpu.CompilerParams(
            dimension_semantics=("arbitrary",),
        ),
    )(te, s1, s2, x_sorted, w13_weight, w2_weight, w_rows)


def _combine_body(down_hbm, pos_hbm, out_hbm, pa_v, pb_v, bufa_v, bufb_v, sem):
    wid = lax.axis_index("s")
    base_t = wid * TOK_PER_TILE
    ctok = CHUNK // TOP_K
    for c in range(XCHUNKS):
        pltpu.sync_copy(pos_hbm.at[pl.ds(base_t + c * ctok, ctok)], pa_v.at[c])
        pltpu.sync_copy(
            pos_hbm.at[pl.ds(TOKENS + base_t + c * ctok, ctok)], pb_v.at[c])
        pltpu.async_copy(down_hbm.at[pa_v.at[c]], bufa_v, sem).wait()
        pltpu.async_copy(down_hbm.at[pb_v.at[c]], bufb_v, sem).wait()

        def hb_body(hb, _):
            sl = pl.ds(hb * L, L)
            for i in range(ctok):
                bufa_v[i, sl] = bufa_v[i, sl] + bufb_v[i, sl]
            return 0

        lax.fori_loop(0, HIDDEN // L, hb_body, 0)
        pltpu.sync_copy(bufa_v, out_hbm.at[pl.ds(base_t + c * ctok, ctok)])


def _combine(down, pos):
    mesh = plsc.VectorSubcoreMesh(core_axis_name="c", subcore_axis_name="s",
                                  num_cores=1)
    ctok = CHUNK // TOP_K
    kern = functools.partial(
        pl.kernel,
        out_type=jax.ShapeDtypeStruct((TOKENS, HIDDEN), _f32),
        mesh=mesh,
        scratch_types=[
            pltpu.VMEM((XCHUNKS, ctok), _i32),
            pltpu.VMEM((XCHUNKS, ctok), _i32),
            pltpu.VMEM((ctok, HIDDEN), _f32),
            pltpu.VMEM((ctok, HIDDEN), _f32),
            pltpu.SemaphoreType.DMA,
        ],
        compiler_params=pltpu.CompilerParams(needs_layout_passes=False),
    )(_combine_body)
    return kern(down, pos)


def kernel(hidden_states, router_logits, w13_weight, w2_weight,
           w13_input_scale, w2_input_scale, w13_weight_scale, w2_weight_scale):
    s1 = (w13_input_scale * w13_weight_scale).astype(_f32)
    s2 = (w2_input_scale * w2_weight_scale).astype(_f32)
    x_sorted, w_sorted, pos, te = _route(
        router_logits.astype(_f32).reshape(TOKENS * NUM_EXPERTS),
        hidden_states.astype(_f32))
    w_rows = w_sorted.reshape(NT, 1, TM)
    down = _gemm(te, s1, s2, x_sorted, w13_weight, w2_weight, w_rows)
    return _combine(down, pos)


# speedup vs baseline: 1.2614x; 1.0271x over previous
import functools

import jax
import jax.numpy as jnp
from jax import lax
from jax.experimental import pallas as pl
from jax.experimental.pallas import tpu as pltpu
from jax.experimental.pallas import tpu_sc as plsc

NUM_EXPERTS = 8
TOP_K = 2
HIDDEN = 1024
INTER = 1024
TOKENS = 2048

L = 16
NTILES = 16
TOK_PER_TILE = TOKENS // NTILES
GROUPS = TOK_PER_TILE // L
TM = 512
PAD_N = TOKENS * TOP_K + NUM_EXPERTS * TM
NT = PAD_N // TM
NT_PAD = 16
XCHUNKS = 4
CHUNK = TOK_PER_TILE * TOP_K // XCHUNKS

_f32 = jnp.float32
_i32 = jnp.int32


def _lanes():
    return lax.iota(_i32, L)


def _splat(vec, lane):
    return jnp.sum(jnp.where(_lanes() == lane, vec, 0))


def _route_body(logits_hbm, hidden_hbm,
                x_sorted_hbm, w_sorted_hbm, pos_hbm, te_hbm,
                lg_v, e_v, w_v, posk_v, slots_v, tok_v, wvals_v,
                cnt_v, allcnt_v, sm_v, te_v, rows_v,
                counts_sh, wsort_sh, sem):
    wid = lax.axis_index("s")
    base_t = wid * TOK_PER_TILE
    lanes = _lanes()

    pltpu.sync_copy(
        logits_hbm.at[pl.ds(base_t * NUM_EXPERTS, TOK_PER_TILE * NUM_EXPERTS)],
        lg_v)
    cnt = jnp.zeros((L,), _i32)
    for g in range(GROUPS):
        row_idx = (lanes + g * L) * NUM_EXPERTS
        cols = [plsc.load_gather(lg_v, [row_idx + e])
                for e in range(NUM_EXPERTS)]
        v1 = jnp.full((L,), -jnp.inf, _f32)
        e1 = jnp.zeros((L,), _i32)
        for e in range(NUM_EXPERTS):
            upd = cols[e] > v1
            v1 = jnp.where(upd, cols[e], v1)
            e1 = jnp.where(upd, e, e1)
        v2 = jnp.full((L,), -jnp.inf, _f32)
        e2 = jnp.zeros((L,), _i32)
        for e in range(NUM_EXPERTS):
            upd = (cols[e] > v2) & (e1 != e)
            v2 = jnp.where(upd, cols[e], v2)
            e2 = jnp.where(upd, e, e2)
        w1 = 1.0 / (1.0 + jnp.exp(v2 - v1))
        w2 = 1.0 - w1
        sl = pl.ds(g * L, L)
        e_v[0, sl] = e1
        e_v[1, sl] = e2
        w_v[0, sl] = w1
        w_v[1, sl] = w2
        for e in range(NUM_EXPERTS):
            pc = (plsc.all_reduce_population_count(e1 == e)
                  + plsc.all_reduce_population_count(e2 == e))
            cnt = cnt + jnp.where(lanes == e, pc, 0)

    cnt_v[...] = cnt
    pltpu.sync_copy(cnt_v, counts_sh.at[wid])
    plsc.subcore_barrier()

    pltpu.sync_copy(counts_sh, allcnt_v)
    total = jnp.zeros((L,), _i32)
    pre = jnp.zeros((L,), _i32)
    for t in range(NTILES):
        row = allcnt_v[t, :]
        total = total + row
        tv = jnp.full((L,), t, _i32)
        wv = jnp.zeros((L,), _i32) + wid
        pre = pre + jnp.where(tv < wv, row, 0)
    pcnt = (total + (TM - 1)) // TM * TM
    pcnt = jnp.where(lanes < NUM_EXPERTS, pcnt, 0)
    pbase = plsc.cumsum(pcnt) - pcnt
    sm_v[...] = pbase + pre

    run = jnp.zeros((L,), _i32)
    for g in range(GROUPS):
        sl = pl.ds(g * L, L)
        for k in range(TOP_K):
            a = e_v[k, sl]
            wt = w_v[k, sl]
            slot = jnp.zeros((L,), _i32)
            sr = sm_v[...] + run
            for e in range(NUM_EXPERTS):
                m = a == e
                r = plsc.cumsum(m.astype(_i32)) - 1
                base_e = _splat(sr, e)
                slot = jnp.where(m, base_e + r, slot)
                run = run + jnp.where(lanes == e, plsc.all_reduce_population_count(m), 0)
            posk_v[k, sl] = slot
            ai = g * (TOP_K * L) + k * L
            c, off = ai // CHUNK, ai % CHUNK
            slots_v[c, pl.ds(off, L)] = slot
            tok_v[c, pl.ds(off, L)] = base_t + g * L + lanes
            wvals_v[c, pl.ds(off, L)] = wt

    pltpu.sync_copy(posk_v.at[0], pos_hbm.at[pl.ds(base_t, TOK_PER_TILE)])
    pltpu.sync_copy(posk_v.at[1], pos_hbm.at[pl.ds(TOKENS + base_t, TOK_PER_TILE)])

    for c in range(XCHUNKS):
        pltpu.sync_copy(wvals_v.at[c], wsort_sh.at[slots_v.at[c]])
    plsc.subcore_barrier()

    @pl.when(wid == 0)
    def _write_meta():
        pltpu.sync_copy(wsort_sh, w_sorted_hbm)
        total_pad = jnp.sum(pcnt)
        for b in range(NT_PAD // L):
            j = lanes + b * L
            jtm = j * TM
            acc = jnp.zeros((L,), _i32)
            for e in range(1, NUM_EXPERTS):
                acc = acc + (jtm >= _splat(pbase, e)).astype(_i32)
            te = jnp.where(jtm < total_pad, acc, 0)
            te_v[pl.ds(b * L, L)] = te
        pltpu.sync_copy(te_v, te_hbm)

    for c in range(XCHUNKS):
        pltpu.async_copy(hidden_hbm.at[tok_v.at[c]], rows_v, sem).wait()
        pltpu.async_copy(rows_v, x_sorted_hbm.at[slots_v.at[c]], sem).wait()


def _route(router_logits, hidden_states):
    mesh = plsc.VectorSubcoreMesh(core_axis_name="c", subcore_axis_name="s",
                                  num_cores=1)
    kern = functools.partial(
        pl.kernel,
        out_type=(
            jax.ShapeDtypeStruct((PAD_N, HIDDEN), _f32),
            jax.ShapeDtypeStruct((PAD_N,), _f32),
            jax.ShapeDtypeStruct((TOP_K * TOKENS,), _i32),
            jax.ShapeDtypeStruct((NT_PAD,), _i32),
        ),
        mesh=mesh,
        scratch_types=[
            pltpu.VMEM((TOK_PER_TILE * NUM_EXPERTS,), _f32),
            pltpu.VMEM((TOP_K, TOK_PER_TILE), _i32),
            pltpu.VMEM((TOP_K, TOK_PER_TILE), _f32),
            pltpu.VMEM((TOP_K, TOK_PER_TILE), _i32),
            pltpu.VMEM((XCHUNKS, CHUNK), _i32),
            pltpu.VMEM((XCHUNKS, CHUNK), _i32),
            pltpu.VMEM((XCHUNKS, CHUNK), _f32),
            pltpu.VMEM((L,), _i32),
            pltpu.VMEM((NTILES, L), _i32),
            pltpu.VMEM((L,), _i32),
            pltpu.VMEM((NT_PAD,), _i32),
            pltpu.VMEM((CHUNK, HIDDEN), _f32),
            pltpu.VMEM_SHARED((NTILES, L), _i32),
            pltpu.VMEM_SHARED((PAD_N,), _f32),
            pltpu.SemaphoreType.DMA,
        ],
        compiler_params=pltpu.CompilerParams(needs_layout_passes=False),
    )(_route_body)
    return kern(router_logits, hidden_states)


def _gemm_body(te_ref, s1_ref, s2_ref, x_ref, w13_ref, w2_ref, wrow_ref,
               out_ref):
    j = pl.program_id(0)
    e = te_ref[j]
    x = x_ref[...]
    w13 = w13_ref[0]
    w2 = w2_ref[0]
    gateup = lax.dot_general(x, w13, (((1,), (1,)), ((), ())),
                             preferred_element_type=_f32)
    gateup = gateup * s1_ref[e]
    gate = gateup[:, :INTER]
    up = gateup[:, INTER:]
    act = gate * (1.0 / (1.0 + jnp.exp(-gate))) * up
    down = lax.dot_general(act, w2, (((1,), (1,)), ((), ())),
                           preferred_element_type=_f32)
    down = down * s2_ref[e]
    out_ref[...] = down * wrow_ref[0, 0, :][:, None]


def _gemm(te, s1, s2, x_sorted, w13_weight, w2_weight, w_rows):
    return pl.pallas_call(
        _gemm_body,
        grid_spec=pltpu.PrefetchScalarGridSpec(
            num_scalar_prefetch=3,
            grid=(NT,),
            in_specs=[
                pl.BlockSpec((TM, HIDDEN), lambda j, te, s1, s2: (j, 0)),
                pl.BlockSpec((1, 2 * INTER, HIDDEN),
                             lambda j, te, s1, s2: (te[j], 0, 0)),
                pl.BlockSpec((1, HIDDEN, INTER),
                             lambda j, te, s1, s2: (te[j], 0, 0)),
                pl.BlockSpec((1, 1, TM), lambda j, te, s1, s2: (j, 0, 0)),
            ],
            out_specs=pl.BlockSpec((TM, HIDDEN), lambda j, te, s1, s2: (j, 0)),
        ),
        out_shape=jax.ShapeDtypeStruct((PAD_N, HIDDEN), _f32),
        compiler_params=pltpu.CompilerParams(
            dimension_semantics=("arbitrary",),
        ),
    )(te, s1, s2, x_sorted, w13_weight, w2_weight, w_rows)


def _combine_body(down_hbm, pos_hbm, out_hbm, pa_v, pb_v, bufa_v, bufb_v, sem):
    wid = lax.axis_index("s")
    base_t = wid * TOK_PER_TILE
    ctok = CHUNK // TOP_K
    for c in range(XCHUNKS):
        pltpu.sync_copy(pos_hbm.at[pl.ds(base_t + c * ctok, ctok)], pa_v.at[c])
        pltpu.sync_copy(
            pos_hbm.at[pl.ds(TOKENS + base_t + c * ctok, ctok)], pb_v.at[c])
        pltpu.async_copy(down_hbm.at[pa_v.at[c]], bufa_v, sem).wait()
        pltpu.async_copy(down_hbm.at[pb_v.at[c]], bufb_v, sem).wait()

        def hb_body(hb, _):
            sl = pl.ds(hb * L, L)
            for i in range(ctok):
                bufa_v[i, sl] = bufa_v[i, sl] + bufb_v[i, sl]
            return 0

        lax.fori_loop(0, HIDDEN // L, hb_body, 0)
        pltpu.sync_copy(bufa_v, out_hbm.at[pl.ds(base_t + c * ctok, ctok)])


def _combine(down, pos):
    mesh = plsc.VectorSubcoreMesh(core_axis_name="c", subcore_axis_name="s",
                                  num_cores=1)
    ctok = CHUNK // TOP_K
    kern = functools.partial(
        pl.kernel,
        out_type=jax.ShapeDtypeStruct((TOKENS, HIDDEN), _f32),
        mesh=mesh,
        scratch_types=[
            pltpu.VMEM((XCHUNKS, ctok), _i32),
            pltpu.VMEM((XCHUNKS, ctok), _i32),
            pltpu.VMEM((ctok, HIDDEN), _f32),
            pltpu.VMEM((ctok, HIDDEN), _f32),
            pltpu.SemaphoreType.DMA,
        ],
        compiler_params=pltpu.CompilerParams(needs_layout_passes=False),
    )(_combine_body)
    return kern(down, pos)


def kernel(hidden_states, router_logits, w13_weight, w2_weight,
           w13_input_scale, w2_input_scale, w13_weight_scale, w2_weight_scale):
    s1 = (w13_input_scale * w13_weight_scale).astype(_f32)
    s2 = (w2_input_scale * w2_weight_scale).astype(_f32)
    x_sorted, w_sorted, pos, te = _route(
        router_logits.astype(_f32).reshape(TOKENS * NUM_EXPERTS),
        hidden_states.astype(_f32))
    w_rows = w_sorted.reshape(NT, 1, TM)
    down = _gemm(te, s1, s2, x_sorted, w13_weight, w2_weight, w_rows)
    return _combine(down, pos)
